# Initial kernel scaffold; baseline (speedup 1.0000x reference)
import jax
import jax.numpy as jnp
from jax.experimental import pallas as pl


def kernel(idx, table):
    raise NotImplementedError("write your pallas kernel here")


# speedup vs baseline: 1.0140x; 1.0140x over previous
import functools

import jax
import jax.numpy as jnp
from jax import lax
from jax.experimental import pallas as pl
from jax.experimental.pallas import tpu as pltpu
from jax.experimental.pallas import tpu_sc as plsc

D = 1000
NC = 2
NS = 16
NW = NC * NS
B_TOTAL = 1024 * 50
B_PER_W = B_TOTAL // NW
CHUNK = 64
N_CHUNKS = B_PER_W // CHUNK

_mesh = plsc.VectorSubcoreMesh(core_axis_name="c", subcore_axis_name="s")


@functools.partial(
    pl.kernel,
    mesh=_mesh,
    out_type=jax.ShapeDtypeStruct((B_TOTAL, D), jnp.float32),
    scratch_types=[
        pltpu.VMEM((B_PER_W,), jnp.int32),
        pltpu.VMEM((CHUNK, D), jnp.float32),
        pltpu.SemaphoreType.DMA,
    ],
    compiler_params=pltpu.CompilerParams(use_tc_tiling_on_sc=False),
)
def _gather_rows(idx_hbm, table_hbm, out_hbm, idx_v, rows_v, sem):
    wid = lax.axis_index("s") * NC + lax.axis_index("c")
    base = wid * B_PER_W
    pltpu.sync_copy(idx_hbm.at[pl.ds(base, B_PER_W)], idx_v)

    def body(i, carry):
        pltpu.async_copy(
            table_hbm.at[idx_v.at[pl.ds(i * CHUNK, CHUNK)]], rows_v, sem
        ).wait()
        pltpu.sync_copy(rows_v, out_hbm.at[pl.ds(base + i * CHUNK, CHUNK)])
        return carry

    lax.fori_loop(0, N_CHUNKS, body, 0)


def kernel(idx, table):
    flat = idx.reshape(-1).astype(jnp.int32)
    out = _gather_rows(flat, table)
    return out.reshape(idx.shape[0], idx.shape[1], D)


# speedup vs baseline: 1.0209x; 1.0068x over previous
import functools

import jax
import jax.numpy as jnp
from jax import lax
from jax.experimental import pallas as pl
from jax.experimental.pallas import tpu as pltpu
from jax.experimental.pallas import tpu_sc as plsc

D = 1000
NC = 2
NS = 16
NW = NC * NS
B_TOTAL = 1024 * 50
B_PER_W = B_TOTAL // NW
CHUNK = 40
N_CHUNKS = B_PER_W // CHUNK
N_PAIRS = N_CHUNKS // 2

_mesh = plsc.VectorSubcoreMesh(core_axis_name="c", subcore_axis_name="s")


@functools.partial(
    pl.kernel,
    mesh=_mesh,
    out_type=jax.ShapeDtypeStruct((B_TOTAL, D), jnp.float32),
    scratch_types=[
        pltpu.VMEM((B_PER_W,), jnp.int32),
        pltpu.VMEM((CHUNK, D), jnp.float32),
        pltpu.VMEM((CHUNK, D), jnp.float32),
        pltpu.SemaphoreType.DMA,
        pltpu.SemaphoreType.DMA,
        pltpu.SemaphoreType.DMA,
        pltpu.SemaphoreType.DMA,
    ],
    compiler_params=pltpu.CompilerParams(use_tc_tiling_on_sc=False),
)
def _gather_rows(idx_hbm, table_hbm, out_hbm, idx_v, buf_a, buf_b,
                 gsem_a, gsem_b, wsem_a, wsem_b):
    wid = lax.axis_index("s") * NC + lax.axis_index("c")
    base = wid * B_PER_W
    pltpu.sync_copy(idx_hbm.at[pl.ds(base, B_PER_W)], idx_v)

    def g_start(i, buf, sem):
        pltpu.async_copy(table_hbm.at[idx_v.at[pl.ds(i * CHUNK, CHUNK)]], buf, sem)

    def g_wait(i, buf, sem):
        pltpu.make_async_copy(
            table_hbm.at[idx_v.at[pl.ds(i * CHUNK, CHUNK)]], buf, sem
        ).wait()

    def w_start(i, buf, sem):
        pltpu.async_copy(buf, out_hbm.at[pl.ds(base + i * CHUNK, CHUNK)], sem)

    def w_wait(i, buf, sem):
        pltpu.make_async_copy(
            buf, out_hbm.at[pl.ds(base + i * CHUNK, CHUNK)], sem
        ).wait()

    g_start(0, buf_a, gsem_a)
    g_start(1, buf_b, gsem_b)

    def pair_body(p, carry):
        i = 2 * p
        g_wait(i, buf_a, gsem_a)
        w_start(i, buf_a, wsem_a)
        g_wait(i + 1, buf_b, gsem_b)
        w_start(i + 1, buf_b, wsem_b)
        w_wait(i, buf_a, wsem_a)
        g_start(i + 2, buf_a, gsem_a)
        w_wait(i + 1, buf_b, wsem_b)
        g_start(i + 3, buf_b, gsem_b)
        return carry

    lax.fori_loop(0, N_PAIRS - 1, pair_body, 0)

    i = N_CHUNKS - 2
    g_wait(i, buf_a, gsem_a)
    w_start(i, buf_a, wsem_a)
    g_wait(i + 1, buf_b, gsem_b)
    w_start(i + 1, buf_b, wsem_b)
    w_wait(i, buf_a, wsem_a)
    w_wait(i + 1, buf_b, wsem_b)


def kernel(idx, table):
    flat = idx.reshape(-1).astype(jnp.int32)
    out = _gather_rows(flat, table)
    return out.reshape(idx.shape[0], idx.shape[1], D)
